# Initial kernel scaffold; baseline (speedup 1.0000x reference)
import jax
import jax.numpy as jnp
from jax.experimental import pallas as pl


def kernel(xyz, feat, W, b, gamma, beta):
    raise NotImplementedError("write your pallas kernel here")


# speedup vs baseline: 4.2953x; 4.2953x over previous
import functools

import jax
import jax.numpy as jnp
from jax import lax
from jax.experimental import pallas as pl
from jax.experimental.pallas import tpu as pltpu
from jax.experimental.pallas import tpu_sc as plsc

_B, _N, _F, _K = 4, 1024, 1024, 8
_EPS = 1e-5

_NC, _NS = 2, 16
_NW = _NC * _NS
_RW = (_B * _N) // _NW
_G = 8
_NCHUNK = _RW // _G
_RBLK = 256


def _topk_body(xyn_ref, xyt_ref, idx_ref):
    xyn = xyn_ref[0]
    xyt = xyt_ref[0]
    inner = 2.0 * lax.dot_general(
        xyn, xyt, (((1,), (0,)), ((), ())),
        preferred_element_type=jnp.float32)
    xx_row = jnp.sum(xyt * xyt, axis=0, keepdims=True)
    xx_col = jnp.sum(xyn * xyn, axis=1, keepdims=True)
    pw = (xx_row - inner) + xx_col

    colids = lax.broadcasted_iota(jnp.int32, (_RBLK, _N), 1)
    cur = pw
    picks = []
    for _ in range(_K):
        m = jnp.min(cur, axis=1, keepdims=True)
        cand = jnp.where(cur == m, colids, _N)
        sel = jnp.min(cand, axis=1, keepdims=True)
        picks.append(sel)
        cur = jnp.where(colids == sel, jnp.inf, cur)
    idx_ref[0] = jnp.concatenate(picks, axis=1)


def _transpose_body(x_ref, o_ref):
    o_ref[0] = jnp.transpose(x_ref[0], (1, 0))


def _gather_mean_body(table_hbm, idx_hbm, out_hbm, idx_v, rows_v, acc_v, sem):
    wid = lax.axis_index("s") * _NC + lax.axis_index("c")
    pltpu.sync_copy(idx_hbm.at[wid], idx_v)
    base = wid * _RW

    def chunk(c, carry):
        pltpu.async_copy(table_hbm.at[idx_v.at[c]], rows_v, sem).wait()

        def jbody(j, carry2):
            col = j * 16
            for g in range(_G):
                s = rows_v[g * _K, pl.ds(col, 16)]
                for k in range(1, _K):
                    s = s + rows_v[g * _K + k, pl.ds(col, 16)]
                acc_v[g, pl.ds(col, 16)] = s * (1.0 / _K)
            return carry2

        lax.fori_loop(0, _F // 16, jbody, 0)
        pltpu.sync_copy(acc_v, out_hbm.at[pl.ds(base + c * _G, _G)])
        return carry

    lax.fori_loop(0, _NCHUNK, chunk, 0)


def _mm_body(feat_ref, m_ref, w_ref, bias_ref, trans_ref, s1_ref, s2_ref):
    lap = feat_ref[0] - m_ref[0]
    t = lax.dot_general(
        lap, w_ref[...], (((1,), (1,)), ((), ())),
        preferred_element_type=jnp.float32) + bias_ref[...]
    trans_ref[0] = t
    s1_ref[0] = jnp.sum(t, axis=1, keepdims=True)
    s2_ref[0] = jnp.sum(t * t, axis=1, keepdims=True)


def _bn_body(trans_ref, feat_ref, s1_ref, s2_ref, g_ref, be_ref, out_ref):
    cnt = float(_B * _F)
    mean = jnp.sum(s1_ref[...], axis=0) / cnt
    var = jnp.sum(s2_ref[...], axis=0) / cnt - mean * mean
    inv = lax.rsqrt(var + _EPS)
    xn = (trans_ref[0] - mean) * inv
    y = xn * g_ref[...] + be_ref[...]
    out_ref[0] = feat_ref[0] + jnp.maximum(y, 0.0)


def _topk_call(xyn, xyt):
    return pl.pallas_call(
        _topk_body,
        grid=(_B, _N // _RBLK),
        in_specs=[
            pl.BlockSpec((1, _RBLK, 2), lambda i, j: (i, j, 0)),
            pl.BlockSpec((1, 2, _N), lambda i, j: (i, 0, 0)),
        ],
        out_specs=pl.BlockSpec((1, _RBLK, _K), lambda i, j: (i, j, 0)),
        out_shape=jax.ShapeDtypeStruct((_B, _N, _K), jnp.int32),
    )(xyn, xyt)


def _transpose_call(feat):
    return pl.pallas_call(
        _transpose_body,
        grid=(_B,),
        in_specs=[pl.BlockSpec((1, _N, _F), lambda i: (i, 0, 0))],
        out_specs=pl.BlockSpec((1, _N, _F), lambda i: (i, 0, 0)),
        out_shape=jax.ShapeDtypeStruct((_B, _N, _F), jnp.float32),
    )(feat)


@functools.lru_cache(maxsize=1)
def _make_gather_mean():
    return functools.partial(
        pl.kernel,
        mesh=plsc.VectorSubcoreMesh(core_axis_name="c", subcore_axis_name="s"),
        out_type=jax.ShapeDtypeStruct((_B * _N, _F), jnp.float32),
        scratch_types=[
            pltpu.VMEM((_NCHUNK, _G * _K), jnp.int32),
            pltpu.VMEM((_G * _K, _F), jnp.float32),
            pltpu.VMEM((_G, _F), jnp.float32),
            pltpu.SemaphoreType.DMA,
        ],
    )(_gather_mean_body)


def _mm_call(feat, m, w, bias):
    return pl.pallas_call(
        _mm_body,
        grid=(_B, _N // _RBLK),
        in_specs=[
            pl.BlockSpec((1, _RBLK, _F), lambda i, j: (i, j, 0)),
            pl.BlockSpec((1, _RBLK, _F), lambda i, j: (i, j, 0)),
            pl.BlockSpec((_F, _F), lambda i, j: (0, 0)),
            pl.BlockSpec((1, _F), lambda i, j: (0, 0)),
        ],
        out_specs=[
            pl.BlockSpec((1, _RBLK, _F), lambda i, j: (i, j, 0)),
            pl.BlockSpec((1, _RBLK, 1), lambda i, j: (i, j, 0)),
            pl.BlockSpec((1, _RBLK, 1), lambda i, j: (i, j, 0)),
        ],
        out_shape=[
            jax.ShapeDtypeStruct((_B, _N, _F), jnp.float32),
            jax.ShapeDtypeStruct((_B, _N, 1), jnp.float32),
            jax.ShapeDtypeStruct((_B, _N, 1), jnp.float32),
        ],
    )(feat, m, w, bias)


def _bn_call(trans, feat, s1, s2, gamma, beta):
    return pl.pallas_call(
        _bn_body,
        grid=(_B, _N // _RBLK),
        in_specs=[
            pl.BlockSpec((1, _RBLK, _F), lambda i, j: (i, j, 0)),
            pl.BlockSpec((1, _RBLK, _F), lambda i, j: (i, j, 0)),
            pl.BlockSpec((_B, _RBLK, 1), lambda i, j: (0, j, 0)),
            pl.BlockSpec((_B, _RBLK, 1), lambda i, j: (0, j, 0)),
            pl.BlockSpec((_RBLK, 1), lambda i, j: (j, 0)),
            pl.BlockSpec((_RBLK, 1), lambda i, j: (j, 0)),
        ],
        out_specs=pl.BlockSpec((1, _RBLK, _F), lambda i, j: (i, j, 0)),
        out_shape=jax.ShapeDtypeStruct((_B, _N, _F), jnp.float32),
    )(trans, feat, s1, s2, gamma, beta)


def kernel(xyz, feat, W, b, gamma, beta):
    xyn = xyz[:, :, :2]
    xyt = jnp.transpose(xyn, (0, 2, 1))
    idx = _topk_call(xyn, xyt)

    featT = _transpose_call(feat)

    cols = idx.reshape(_B, _K, _N // _K, _K).transpose(0, 2, 3, 1)
    cols = cols.reshape(_B, _N, _K)
    cols = cols + (jnp.arange(_B, dtype=jnp.int32) * _N)[:, None, None]
    idx_sc = cols.reshape(_NW, _NCHUNK, _G * _K)

    m = _make_gather_mean()(featT.reshape(_B * _N, _F), idx_sc)
    m = m.reshape(_B, _N, _F)

    trans, s1, s2 = _mm_call(feat, m, W, b.reshape(1, _F))
    out = _bn_call(trans, feat, s1, s2,
                   gamma.reshape(_N, 1), beta.reshape(_N, 1))
    return out


# speedup vs baseline: 4.7195x; 1.0987x over previous
import functools

import jax
import jax.numpy as jnp
from jax import lax
from jax.experimental import pallas as pl
from jax.experimental.pallas import tpu as pltpu
from jax.experimental.pallas import tpu_sc as plsc

_B, _N, _F, _K = 4, 1024, 1024, 8
_EPS = 1e-5

_NC, _NS = 2, 16
_NW = _NC * _NS
_RW = (_B * _N) // _NW
_G = 4
_NCHUNK = _RW // _G
_RBLK = 256


def _topk_body(xyn_ref, xyt_ref, feat_ref, idx_ref, featT_ref):
    xyn = xyn_ref[0]
    xyt = xyt_ref[0]
    inner = 2.0 * lax.dot_general(
        xyn, xyt, (((1,), (0,)), ((), ())),
        preferred_element_type=jnp.float32)
    xx_row = jnp.sum(xyt * xyt, axis=0, keepdims=True)
    xx_col = jnp.sum(xyn * xyn, axis=1, keepdims=True)
    pw = (xx_row - inner) + xx_col

    colids = lax.broadcasted_iota(jnp.int32, (_RBLK, _N), 1)
    cur = pw
    picks = []
    for _ in range(_K):
        m = jnp.min(cur, axis=1, keepdims=True)
        cand = jnp.where(cur == m, colids, _N)
        sel = jnp.min(cand, axis=1, keepdims=True)
        picks.append(sel)
        cur = jnp.where(colids == sel, jnp.inf, cur)
    idx_ref[0] = jnp.concatenate(picks, axis=1)

    featT_ref[0] = jnp.transpose(feat_ref[0], (1, 0))


def _topk_call(xyn, xyt, feat):
    return pl.pallas_call(
        _topk_body,
        grid=(_B, _N // _RBLK),
        in_specs=[
            pl.BlockSpec((1, _RBLK, 2), lambda i, j: (i, j, 0)),
            pl.BlockSpec((1, 2, _N), lambda i, j: (i, 0, 0)),
            pl.BlockSpec((1, _RBLK, _F), lambda i, j: (i, j, 0)),
        ],
        out_specs=[
            pl.BlockSpec((1, _RBLK, _K), lambda i, j: (i, j, 0)),
            pl.BlockSpec((1, _F, _RBLK), lambda i, j: (i, 0, j)),
        ],
        out_shape=[
            jax.ShapeDtypeStruct((_B, _N, _K), jnp.int32),
            jax.ShapeDtypeStruct((_B, _N, _F), jnp.float32),
        ],
    )(xyn, xyt, feat)


def _gather_mean_body(table_hbm, idx_hbm, out_hbm,
                      idx_v, rows0, rows1, acc0, acc1,
                      sem0, sem1, ws0, ws1):
    wid = lax.axis_index("s") * _NC + lax.axis_index("c")
    pltpu.sync_copy(idx_hbm.at[wid], idx_v)
    base = wid * _RW
    rows = (rows0, rows1)
    acc = (acc0, acc1)
    gsem = (sem0, sem1)
    wsem = (ws0, ws1)

    pltpu.async_copy(table_hbm.at[idx_v.at[0]], rows0, sem0)
    pltpu.async_copy(table_hbm.at[idx_v.at[1]], rows1, sem1)

    def outer(cc, carry):
        for par in range(2):
            c = cc * 2 + par
            pltpu.make_async_copy(
                table_hbm.at[pl.ds(0, _G * _K)], rows[par], gsem[par]).wait()

            @pl.when(c >= 2)
            def _():
                pltpu.make_async_copy(
                    acc[par], out_hbm.at[pl.ds(0, _G)], wsem[par]).wait()

            def jbody(jj, carry2):
                col = jj * 16
                for g in range(_G):
                    r = [rows[par][g * _K + k, pl.ds(col, 16)]
                         for k in range(_K)]
                    s = ((r[0] + r[1]) + (r[2] + r[3])) + \
                        ((r[4] + r[5]) + (r[6] + r[7]))
                    acc[par][g, pl.ds(col, 16)] = s * (1.0 / _K)
                return carry2

            lax.fori_loop(0, _F // 16, jbody, 0)

            @pl.when(c + 2 < _NCHUNK)
            def _():
                pltpu.async_copy(
                    table_hbm.at[idx_v.at[c + 2]], rows[par], gsem[par])

            pltpu.async_copy(
                acc[par], out_hbm.at[pl.ds(base + c * _G, _G)], wsem[par])
        return carry

    lax.fori_loop(0, _NCHUNK // 2, outer, 0)

    pltpu.make_async_copy(acc0, out_hbm.at[pl.ds(0, _G)], ws0).wait()
    pltpu.make_async_copy(acc1, out_hbm.at[pl.ds(0, _G)], ws1).wait()


@functools.lru_cache(maxsize=1)
def _make_gather_mean():
    return functools.partial(
        pl.kernel,
        mesh=plsc.VectorSubcoreMesh(core_axis_name="c", subcore_axis_name="s"),
        out_type=jax.ShapeDtypeStruct((_B * _N, _F), jnp.float32),
        scratch_types=[
            pltpu.VMEM((_NCHUNK, _G * _K), jnp.int32),
            pltpu.VMEM((_G * _K, _F), jnp.float32),
            pltpu.VMEM((_G * _K, _F), jnp.float32),
            pltpu.VMEM((_G, _F), jnp.float32),
            pltpu.VMEM((_G, _F), jnp.float32),
            pltpu.SemaphoreType.DMA,
            pltpu.SemaphoreType.DMA,
            pltpu.SemaphoreType.DMA,
            pltpu.SemaphoreType.DMA,
        ],
    )(_gather_mean_body)


def _fused_mm_bn_body(feat_ref, m_ref, w_ref, bias_ref, g_ref, be_ref,
                      out_ref, trans_s, s1_s, s2_s):
    p = pl.program_id(0)
    bi = pl.program_id(1)
    j = pl.program_id(2)
    sl = pl.ds(j * _RBLK, _RBLK)

    @pl.when(p == 0)
    def _():
        lap = feat_ref[0] - m_ref[0]
        t = lax.dot_general(
            lap, w_ref[...], (((1,), (1,)), ((), ())),
            preferred_element_type=jnp.float32) + bias_ref[...]
        trans_s[bi, sl, :] = t
        rs1 = jnp.sum(t, axis=1, keepdims=True)
        rs2 = jnp.sum(t * t, axis=1, keepdims=True)

        @pl.when(bi == 0)
        def _():
            s1_s[sl] = rs1
            s2_s[sl] = rs2

        @pl.when(bi != 0)
        def _():
            s1_s[sl] = s1_s[sl] + rs1
            s2_s[sl] = s2_s[sl] + rs2

    @pl.when(p == 1)
    def _():
        cnt = float(_B * _F)
        mean = s1_s[sl] / cnt
        var = s2_s[sl] / cnt - mean * mean
        inv = lax.rsqrt(var + _EPS)
        t = trans_s[bi, sl, :]
        y = (t - mean) * inv * g_ref[...] + be_ref[...]
        out_ref[0] = feat_ref[0] + jnp.maximum(y, 0.0)


def _fused_mm_bn_call(feat, m, w, bias, gamma, beta):
    def m_map(p, i, j):
        return (jnp.where(p == 0, i, 0), jnp.where(p == 0, j, 0), 0)

    def out_map(p, i, j):
        return (jnp.where(p == 0, 0, i), jnp.where(p == 0, 0, j), 0)

    return pl.pallas_call(
        _fused_mm_bn_body,
        grid=(2, _B, _N // _RBLK),
        in_specs=[
            pl.BlockSpec((1, _RBLK, _F), lambda p, i, j: (i, j, 0)),
            pl.BlockSpec((1, _RBLK, _F), m_map),
            pl.BlockSpec((_F, _F), lambda p, i, j: (0, 0)),
            pl.BlockSpec((1, _F), lambda p, i, j: (0, 0)),
            pl.BlockSpec((_RBLK, 1), lambda p, i, j: (j, 0)),
            pl.BlockSpec((_RBLK, 1), lambda p, i, j: (j, 0)),
        ],
        out_specs=pl.BlockSpec((1, _RBLK, _F), out_map),
        out_shape=jax.ShapeDtypeStruct((_B, _N, _F), jnp.float32),
        scratch_shapes=[
            pltpu.VMEM((_B, _N, _F), jnp.float32),
            pltpu.VMEM((_N, 1), jnp.float32),
            pltpu.VMEM((_N, 1), jnp.float32),
        ],
    )(feat, m, w, bias, gamma, beta)


def kernel(xyz, feat, W, b, gamma, beta):
    xyn = xyz[:, :, :2]
    xyt = jnp.transpose(xyn, (0, 2, 1))
    idx, featT = _topk_call(xyn, xyt, feat)

    cols = idx.reshape(_B, _K, _N // _K, _K).transpose(0, 2, 3, 1)
    cols = cols.reshape(_B, _N, _K)
    cols = cols + (jnp.arange(_B, dtype=jnp.int32) * _N)[:, None, None]
    idx_sc = cols.reshape(_NW, _NCHUNK, _G * _K)

    m = _make_gather_mean()(featT.reshape(_B * _N, _F), idx_sc)
    m = m.reshape(_B, _N, _F)

    return _fused_mm_bn_call(feat, m, W, b.reshape(1, _F),
                             gamma.reshape(_N, 1), beta.reshape(_N, 1))


# speedup vs baseline: 5.9506x; 1.2609x over previous
import functools

import jax
import jax.numpy as jnp
from jax import lax
from jax.experimental import pallas as pl
from jax.experimental.pallas import tpu as pltpu
from jax.experimental.pallas import tpu_sc as plsc

_B, _N, _F, _K = 4, 1024, 1024, 8
_EPS = 1e-5

_NC, _NS = 2, 16
_NW = _NC * _NS
_RW = (_B * _N) // _NW
_G = 4
_NCHUNK = _RW // _G
_RBLK = 256


def _topk_body(xyn_ref, xyt_ref, feat_ref, idx_ref, featT_ref):
    xyn = xyn_ref[0]
    xyt = xyt_ref[0]
    inner = 2.0 * lax.dot_general(
        xyn, xyt, (((1,), (0,)), ((), ())),
        preferred_element_type=jnp.float32)
    xx_row = jnp.sum(xyt * xyt, axis=0, keepdims=True)
    xx_col = jnp.sum(xyn * xyn, axis=1, keepdims=True)
    pw = (xx_row - inner) + xx_col

    colids = lax.broadcasted_iota(jnp.int32, (_RBLK, _N), 1)
    cur = pw
    picks = []
    for _ in range(_K):
        m = jnp.min(cur, axis=1, keepdims=True)
        cand = jnp.where(cur == m, colids, _N)
        sel = jnp.min(cand, axis=1, keepdims=True)
        picks.append(sel)
        cur = jnp.where(colids == sel, jnp.inf, cur)
    idx_ref[0] = jnp.concatenate(picks, axis=1)

    featT_ref[0] = jnp.transpose(feat_ref[0], (1, 0))


def _topk_call(xyn, xyt, feat):
    return pl.pallas_call(
        _topk_body,
        grid=(_B, _N // _RBLK),
        in_specs=[
            pl.BlockSpec((1, _RBLK, 2), lambda i, j: (i, j, 0)),
            pl.BlockSpec((1, 2, _N), lambda i, j: (i, 0, 0)),
            pl.BlockSpec((1, _RBLK, _F), lambda i, j: (i, j, 0)),
        ],
        out_specs=[
            pl.BlockSpec((1, _RBLK, _K), lambda i, j: (i, j, 0)),
            pl.BlockSpec((1, _F, _RBLK), lambda i, j: (i, 0, j)),
        ],
        out_shape=[
            jax.ShapeDtypeStruct((_B, _N, _K), jnp.int32),
            jax.ShapeDtypeStruct((_B, _N, _F), jnp.float32),
        ],
    )(xyn, xyt, feat)


def _gather_mean_body(table_hbm, idx_hbm, out_hbm,
                      idx_v, rows0, rows1, acc0, acc1,
                      sem0, sem1, ws0, ws1):
    wid = lax.axis_index("s") * _NC + lax.axis_index("c")
    pltpu.sync_copy(idx_hbm.at[wid], idx_v)
    base = wid * _RW
    rows = (rows0, rows1)
    acc = (acc0, acc1)
    gsem = (sem0, sem1)
    wsem = (ws0, ws1)

    pltpu.async_copy(table_hbm.at[idx_v.at[0]], rows0, sem0)
    pltpu.async_copy(table_hbm.at[idx_v.at[1]], rows1, sem1)

    def outer(cc, carry):
        for par in range(2):
            c = cc * 2 + par
            pltpu.make_async_copy(
                table_hbm.at[pl.ds(0, _G * _K)], rows[par], gsem[par]).wait()

            @pl.when(c >= 2)
            def _():
                pltpu.make_async_copy(
                    acc[par], out_hbm.at[pl.ds(0, _G)], wsem[par]).wait()

            @plsc.parallel_loop(0, _F // 16, unroll=2)
            def _(jj):
                col = jj * 16
                for g in range(_G):
                    r = [rows[par][g * _K + k, pl.ds(col, 16)]
                         for k in range(_K)]
                    s = ((r[0] + r[1]) + (r[2] + r[3])) + \
                        ((r[4] + r[5]) + (r[6] + r[7]))
                    acc[par][g, pl.ds(col, 16)] = s * (1.0 / _K)

            @pl.when(c + 2 < _NCHUNK)
            def _():
                pltpu.async_copy(
                    table_hbm.at[idx_v.at[c + 2]], rows[par], gsem[par])

            pltpu.async_copy(
                acc[par], out_hbm.at[pl.ds(base + c * _G, _G)], wsem[par])
        return carry

    lax.fori_loop(0, _NCHUNK // 2, outer, 0)

    pltpu.make_async_copy(acc0, out_hbm.at[pl.ds(0, _G)], ws0).wait()
    pltpu.make_async_copy(acc1, out_hbm.at[pl.ds(0, _G)], ws1).wait()


@functools.lru_cache(maxsize=1)
def _make_gather_mean():
    return functools.partial(
        pl.kernel,
        mesh=plsc.VectorSubcoreMesh(core_axis_name="c", subcore_axis_name="s"),
        out_type=jax.ShapeDtypeStruct((_B * _N, _F), jnp.float32),
        scratch_types=[
            pltpu.VMEM((_NCHUNK, _G * _K), jnp.int32),
            pltpu.VMEM((_G * _K, _F), jnp.float32),
            pltpu.VMEM((_G * _K, _F), jnp.float32),
            pltpu.VMEM((_G, _F), jnp.float32),
            pltpu.VMEM((_G, _F), jnp.float32),
            pltpu.SemaphoreType.DMA,
            pltpu.SemaphoreType.DMA,
            pltpu.SemaphoreType.DMA,
            pltpu.SemaphoreType.DMA,
        ],
    )(_gather_mean_body)


def _fused_mm_bn_body(feat_ref, m_ref, w_ref, bias_ref, g_ref, be_ref,
                      out_ref, trans_s, s1_s, s2_s):
    p = pl.program_id(0)
    bi = pl.program_id(1)
    j = pl.program_id(2)
    sl = pl.ds(j * _RBLK, _RBLK)

    @pl.when(p == 0)
    def _():
        lap = feat_ref[0] - m_ref[0]
        t = lax.dot_general(
            lap, w_ref[...], (((1,), (1,)), ((), ())),
            preferred_element_type=jnp.float32) + bias_ref[...]
        trans_s[bi, sl, :] = t
        rs1 = jnp.sum(t, axis=1, keepdims=True)
        rs2 = jnp.sum(t * t, axis=1, keepdims=True)

        @pl.when(bi == 0)
        def _():
            s1_s[sl] = rs1
            s2_s[sl] = rs2

        @pl.when(bi != 0)
        def _():
            s1_s[sl] = s1_s[sl] + rs1
            s2_s[sl] = s2_s[sl] + rs2

    @pl.when(p == 1)
    def _():
        cnt = float(_B * _F)
        mean = s1_s[sl] / cnt
        var = s2_s[sl] / cnt - mean * mean
        inv = lax.rsqrt(var + _EPS)
        t = trans_s[bi, sl, :]
        y = (t - mean) * inv * g_ref[...] + be_ref[...]
        out_ref[0] = feat_ref[0] + jnp.maximum(y, 0.0)


def _fused_mm_bn_call(feat, m, w, bias, gamma, beta):
    def m_map(p, i, j):
        return (jnp.where(p == 0, i, 0), jnp.where(p == 0, j, 0), 0)

    def out_map(p, i, j):
        return (jnp.where(p == 0, 0, i), jnp.where(p == 0, 0, j), 0)

    return pl.pallas_call(
        _fused_mm_bn_body,
        grid=(2, _B, _N // _RBLK),
        in_specs=[
            pl.BlockSpec((1, _RBLK, _F), lambda p, i, j: (i, j, 0)),
            pl.BlockSpec((1, _RBLK, _F), m_map),
            pl.BlockSpec((_F, _F), lambda p, i, j: (0, 0)),
            pl.BlockSpec((1, _F), lambda p, i, j: (0, 0)),
            pl.BlockSpec((_RBLK, 1), lambda p, i, j: (j, 0)),
            pl.BlockSpec((_RBLK, 1), lambda p, i, j: (j, 0)),
        ],
        out_specs=pl.BlockSpec((1, _RBLK, _F), out_map),
        out_shape=jax.ShapeDtypeStruct((_B, _N, _F), jnp.float32),
        scratch_shapes=[
            pltpu.VMEM((_B, _N, _F), jnp.float32),
            pltpu.VMEM((_N, 1), jnp.float32),
            pltpu.VMEM((_N, 1), jnp.float32),
        ],
    )(feat, m, w, bias, gamma, beta)


def kernel(xyz, feat, W, b, gamma, beta):
    xyn = xyz[:, :, :2]
    xyt = jnp.transpose(xyn, (0, 2, 1))
    idx, featT = _topk_call(xyn, xyt, feat)

    cols = idx.reshape(_B, _K, _N // _K, _K).transpose(0, 2, 3, 1)
    cols = cols.reshape(_B, _N, _K)
    cols = cols + (jnp.arange(_B, dtype=jnp.int32) * _N)[:, None, None]
    idx_sc = cols.reshape(_NW, _NCHUNK, _G * _K)

    m = _make_gather_mean()(featT.reshape(_B * _N, _F), idx_sc)
    m = m.reshape(_B, _N, _F)

    return _fused_mm_bn_call(feat, m, W, b.reshape(1, _F),
                             gamma.reshape(_N, 1), beta.reshape(_N, 1))
